# baseline (speedup 1.0000x reference)
import functools

import jax
import jax.numpy as jnp
from jax import lax
from jax.experimental import pallas as pl
from jax.experimental.pallas import tpu as pltpu
from jax.experimental.pallas import tpu_sc as plsc

_B = 128
_N = 32768
_L = 16
_NW = 32
_ROWS = _B // _NW
_NV = _N // _L
_NBINS = 256
_NG = _NBINS // _L
_NPW = _N // 4


def _iota():
    return lax.iota(jnp.int32, _L)


def _extract(v, lane):
    return jnp.sum(jnp.where(_iota() == lane, v, jnp.int32(0)))


def _to_key(xi):
    m = lax.shift_right_arithmetic(xi, 31)
    return lax.bitwise_xor(xi, lax.bitwise_and(m, jnp.int32(0x7FFFFFFF)))


def _clear_hist(hist_v):
    for g in range(_NG):
        hist_v[pl.ds(g * _L, _L)] = jnp.zeros((_L,), jnp.int32)


def _scan_hist(hist_v, t_rem):
    def body(g, c):
        run, found, b_sel, base_sel, cnt_sel = c
        v = hist_v[pl.ds(g * _L, _L)]
        inc = plsc.cumsum(v) + run
        cond = inc > t_rem
        npc = jnp.max(plsc.all_reduce_population_count(cond))
        lane = jnp.int32(_L) - npc
        hit = jnp.logical_and(found == 0, npc > 0)
        inc_at = _extract(inc, lane)
        v_at = _extract(v, lane)
        b_sel = jnp.where(hit, g * _L + lane, b_sel)
        base_sel = jnp.where(hit, inc_at - v_at, base_sel)
        cnt_sel = jnp.where(hit, v_at, cnt_sel)
        found = jnp.where(hit, jnp.int32(1), found)
        run = _extract(inc, jnp.int32(_L - 1))
        return run, found, b_sel, base_sel, cnt_sel

    z = jnp.int32(0)
    _, _, b, base, cnt = lax.fori_loop(0, _NG, body, (z, z, z, z, z))
    return b, base, cnt


def _do_row(out_hbm, row, data_v, cand_v, hist_v, ks_v, sem_out, out_prev):
    k = jnp.max(plsc.load_gather(ks_v, [jnp.full((_L,), row, jnp.int32)]))
    t = k - 1

    _clear_hist(hist_v)

    @plsc.parallel_loop(0, _N, step=_L, unroll=8)
    def _(i):
        x = data_v[pl.ds(i, _L)]
        key = _to_key(plsc.bitcast(x, jnp.int32))
        data_v[pl.ds(i, _L)] = plsc.bitcast(key, jnp.float32)
        d = lax.shift_right_arithmetic(key, 24) + jnp.int32(128)
        cnt, msk = plsc.scan_count(d)
        plsc.addupdate_scatter(hist_v, [d], cnt, mask=msk)

    b0, base0, cnt0 = _scan_hist(hist_v, t)
    t_c = t - base0
    n_less = base0
    t_acc = lax.shift_left(b0 - jnp.int32(128), jnp.int32(24))

    if out_prev is not None:
        out_prev.wait()

    def compact0(i, off_v):
        key = plsc.bitcast(data_v[pl.ds(i, _L)], jnp.int32)
        d = lax.shift_right_arithmetic(key, 24) + jnp.int32(128)
        m = d == b0
        mi = jnp.where(m, jnp.int32(1), jnp.int32(0))
        inc = plsc.cumsum(mi)
        plsc.store_scatter(cand_v, [off_v + inc - mi], key, mask=m)
        return off_v + plsc.all_reduce_population_count(m)

    off_v = plsc.parallel_loop(
        0, _N, step=_L, unroll=8, carry=jnp.zeros((_L,), jnp.int32))(compact0)
    cand_n = jnp.max(off_v)

    for shift in (16, 8, 0):
        _clear_hist(hist_v)
        nvc = lax.div(cand_n + jnp.int32(_L - 1), jnp.int32(_L))

        def h_body(i, carry, shift=shift):
            key = cand_v[pl.ds(i * _L, _L)]
            valid = (i * _L + _iota()) < cand_n
            d = lax.bitwise_and(
                lax.shift_right_arithmetic(key, jnp.int32(shift)),
                jnp.int32(0xFF))
            cnt, msk = plsc.scan_count(d, mask=valid)
            plsc.addupdate_scatter(
                hist_v, [d], cnt, mask=jnp.logical_and(msk, valid))
            return carry

        lax.fori_loop(0, nvc, h_body, jnp.int32(0))

        b, base, cnt = _scan_hist(hist_v, t_c)
        t_c = t_c - base
        n_less = n_less + base
        t_acc = lax.bitwise_or(t_acc, lax.shift_left(b, jnp.int32(shift)))

        if shift != 0:
            def c_body(i, off, shift=shift, b=b, cand_n=cand_n):
                key = cand_v[pl.ds(i * _L, _L)]
                valid = (i * _L + _iota()) < cand_n
                d = lax.bitwise_and(
                    lax.shift_right_arithmetic(key, jnp.int32(shift)),
                    jnp.int32(0xFF))
                m = jnp.logical_and(valid, d == b)
                mi = jnp.where(m, jnp.int32(1), jnp.int32(0))
                inc = plsc.cumsum(mi)
                plsc.store_scatter(cand_v, [off + inc - mi], key, mask=m)
                return off + _extract(inc, jnp.int32(_L - 1))

            cand_n = lax.fori_loop(0, nvc, c_body, jnp.int32(0))
        else:
            cnt3 = cnt

    rem = k - n_less

    @pl.when(cnt3 == 1)
    def _():
        @plsc.parallel_loop(0, _NPW, step=_L, unroll=4)
        def _(i):
            w = jnp.zeros((_L,), jnp.int32)
            for j in range(4):
                idx = 4 * i + 4 * _iota() + jnp.int32(j)
                key = plsc.bitcast(plsc.load_gather(data_v, [idx]), jnp.int32)
                bit = jnp.where(key <= t_acc, jnp.int32(1), jnp.int32(0))
                w = lax.bitwise_or(w, lax.shift_left(bit, jnp.int32(8 * j)))
            cand_v[pl.ds(i, _L)] = w

    @pl.when(cnt3 != 1)
    def _():
        def f_body(i, c_v):
            key = plsc.bitcast(data_v[pl.ds(i * _L, _L)], jnp.int32)
            lt = key < t_acc
            eq = key == t_acc
            eqi = jnp.where(eq, jnp.int32(1), jnp.int32(0))
            inc = plsc.cumsum(eqi)
            take = jnp.logical_and(eq, (c_v + inc - eqi) < rem)
            cand_v[pl.ds(_NPW + i * _L, _L)] = jnp.where(
                jnp.logical_or(lt, take), jnp.int32(1), jnp.int32(0))
            return c_v + plsc.all_reduce_population_count(eq)

        lax.fori_loop(0, _NV, f_body, jnp.zeros((_L,), jnp.int32))

        @plsc.parallel_loop(0, _NPW, step=_L, unroll=4)
        def _(i):
            w = jnp.zeros((_L,), jnp.int32)
            for j in range(4):
                idx = _NPW + 4 * i + 4 * _iota() + jnp.int32(j)
                bit = plsc.load_gather(cand_v, [idx])
                w = lax.bitwise_or(w, lax.shift_left(bit, jnp.int32(8 * j)))
            cand_v[pl.ds(i, _L)] = w

    return pltpu.async_copy(
        cand_v.at[pl.ds(0, _NPW)], out_hbm.at[row], sem_out)


_mesh = plsc.VectorSubcoreMesh(core_axis_name="c", subcore_axis_name="s")


@functools.partial(
    pl.kernel,
    out_type=jax.ShapeDtypeStruct((_B, _NPW), jnp.int32),
    mesh=_mesh,
    compiler_params=pltpu.CompilerParams(needs_layout_passes=False),
    scratch_types=[
        pltpu.VMEM((_N,), jnp.float32),
        pltpu.VMEM((_N,), jnp.float32),
        pltpu.VMEM((_N + _NPW,), jnp.int32),
        pltpu.VMEM((_NBINS,), jnp.int32),
        pltpu.VMEM((_B,), jnp.int32),
        pltpu.SemaphoreType.DMA,
        pltpu.SemaphoreType.DMA,
    ],
)
def _topk_mask_kernel(prior_hbm, ks_hbm, out_hbm, data_a, data_b, cand_v,
                      hist_v, ks_v, sem_in, sem_out):
    cid = lax.axis_index("c")
    sid = lax.axis_index("s")
    wid = sid * 2 + cid
    pltpu.sync_copy(ks_hbm, ks_v)
    bufs = (data_a, data_b)
    row0 = wid * _ROWS
    in_cp = pltpu.async_copy(prior_hbm.at[row0], bufs[0], sem_in)
    out_cp = None
    for r in range(_ROWS):
        in_cp.wait()
        if r + 1 < _ROWS:
            in_cp = pltpu.async_copy(
                prior_hbm.at[row0 + r + 1], bufs[(r + 1) % 2], sem_in)
        out_cp = _do_row(out_hbm, row0 + r, bufs[r % 2], cand_v, hist_v,
                         ks_v, sem_out, out_cp)
    out_cp.wait()


def kernel(prior, rates):
    ks = jnp.clip((_N * rates).astype(jnp.int32), 0, _N).reshape(_B)
    packed = _topk_mask_kernel(prior, ks)
    bytes_ = lax.bitwise_and(
        packed[:, :, None] >> (8 * jnp.arange(4, dtype=jnp.int32)),
        jnp.int32(1))
    return (bytes_ != 0).reshape(_B, _N)


# speedup vs baseline: 1.1457x; 1.1457x over previous
import functools

import jax
import jax.numpy as jnp
from jax import lax
from jax.experimental import pallas as pl
from jax.experimental.pallas import tpu as pltpu
from jax.experimental.pallas import tpu_sc as plsc

_B = 128
_N = 32768
_L = 16
_NW = 32
_ROWS = _B // _NW
_NV = _N // _L
_NBINS = 256
_NG = _NBINS // _L


def _iota():
    return lax.iota(jnp.int32, _L)


def _extract(v, lane):
    return jnp.sum(jnp.where(_iota() == lane, v, jnp.int32(0)))


def _to_key(xi):
    m = lax.shift_right_arithmetic(xi, 31)
    return lax.bitwise_xor(xi, lax.bitwise_and(m, jnp.int32(0x7FFFFFFF)))


def _clear_hist(hist_v):
    for g in range(_NG):
        hist_v[pl.ds(g * _L, _L)] = jnp.zeros((_L,), jnp.int32)


def _scan_hist(hist_v, t_rem):
    def body(g, c):
        run, found, b_sel, base_sel, cnt_sel = c
        v = hist_v[pl.ds(g * _L, _L)]
        inc = plsc.cumsum(v) + run
        cond = inc > t_rem
        npc = jnp.max(plsc.all_reduce_population_count(cond))
        lane = jnp.int32(_L) - npc
        hit = jnp.logical_and(found == 0, npc > 0)
        inc_at = _extract(inc, lane)
        v_at = _extract(v, lane)
        b_sel = jnp.where(hit, g * _L + lane, b_sel)
        base_sel = jnp.where(hit, inc_at - v_at, base_sel)
        cnt_sel = jnp.where(hit, v_at, cnt_sel)
        found = jnp.where(hit, jnp.int32(1), found)
        run = _extract(inc, jnp.int32(_L - 1))
        return run, found, b_sel, base_sel, cnt_sel

    z = jnp.int32(0)
    _, _, b, base, cnt = lax.fori_loop(0, _NG, body, (z, z, z, z, z))
    return b, base, cnt


def _do_row(out_hbm, row, data_v, cand_v, hist_v, ks_v, sem_out, out_prev):
    k = jnp.max(plsc.load_gather(ks_v, [jnp.full((_L,), row, jnp.int32)]))
    t = k - 1

    _clear_hist(hist_v)

    @plsc.parallel_loop(0, _N, step=_L, unroll=16)
    def _(i):
        x = data_v[pl.ds(i, _L)]
        key = _to_key(plsc.bitcast(x, jnp.int32))
        data_v[pl.ds(i, _L)] = plsc.bitcast(key, jnp.float32)
        d = lax.shift_right_arithmetic(key, 24) + jnp.int32(128)
        cnt, msk = plsc.scan_count(d)
        plsc.addupdate_scatter(hist_v, [d], cnt, mask=msk)

    b0, base0, cnt0 = _scan_hist(hist_v, t)
    t_c = t - base0
    n_less = base0
    t_acc = lax.shift_left(b0 - jnp.int32(128), jnp.int32(24))

    if out_prev is not None:
        out_prev.wait()

    def compact0(i, off_v):
        key = plsc.bitcast(data_v[pl.ds(i, _L)], jnp.int32)
        d = lax.shift_right_arithmetic(key, 24) + jnp.int32(128)
        m = d == b0
        mi = jnp.where(m, jnp.int32(1), jnp.int32(0))
        inc = plsc.cumsum(mi)
        plsc.store_scatter(cand_v, [off_v + inc - mi], key, mask=m)
        return off_v + plsc.all_reduce_population_count(m)

    off_v = plsc.parallel_loop(
        0, _N, step=_L, unroll=8, carry=jnp.zeros((_L,), jnp.int32))(compact0)
    cand_n = jnp.max(off_v)

    for shift in (16, 8, 0):
        _clear_hist(hist_v)
        nvc = lax.div(cand_n + jnp.int32(_L - 1), jnp.int32(_L))

        def h_body(i, carry, shift=shift):
            key = cand_v[pl.ds(i * _L, _L)]
            valid = (i * _L + _iota()) < cand_n
            d = lax.bitwise_and(
                lax.shift_right_arithmetic(key, jnp.int32(shift)),
                jnp.int32(0xFF))
            cnt, msk = plsc.scan_count(d, mask=valid)
            plsc.addupdate_scatter(
                hist_v, [d], cnt, mask=jnp.logical_and(msk, valid))
            return carry

        lax.fori_loop(0, nvc, h_body, jnp.int32(0))

        b, base, cnt = _scan_hist(hist_v, t_c)
        t_c = t_c - base
        n_less = n_less + base
        t_acc = lax.bitwise_or(t_acc, lax.shift_left(b, jnp.int32(shift)))

        if shift != 0:
            def c_body(i, off, shift=shift, b=b, cand_n=cand_n):
                key = cand_v[pl.ds(i * _L, _L)]
                valid = (i * _L + _iota()) < cand_n
                d = lax.bitwise_and(
                    lax.shift_right_arithmetic(key, jnp.int32(shift)),
                    jnp.int32(0xFF))
                m = jnp.logical_and(valid, d == b)
                mi = jnp.where(m, jnp.int32(1), jnp.int32(0))
                inc = plsc.cumsum(mi)
                plsc.store_scatter(cand_v, [off + inc - mi], key, mask=m)
                return off + _extract(inc, jnp.int32(_L - 1))

            cand_n = lax.fori_loop(0, nvc, c_body, jnp.int32(0))
        else:
            cnt3 = cnt

    rem = k - n_less

    @pl.when(cnt3 == 1)
    def _():
        @plsc.parallel_loop(0, _N, step=_L, unroll=8)
        def _(i):
            key = plsc.bitcast(data_v[pl.ds(i, _L)], jnp.int32)
            cand_v[pl.ds(i, _L)] = jnp.where(
                key <= t_acc, jnp.int32(1), jnp.int32(0))

    @pl.when(cnt3 != 1)
    def _():
        def f_body(i, c_v):
            key = plsc.bitcast(data_v[pl.ds(i * _L, _L)], jnp.int32)
            lt = key < t_acc
            eq = key == t_acc
            eqi = jnp.where(eq, jnp.int32(1), jnp.int32(0))
            inc = plsc.cumsum(eqi)
            take = jnp.logical_and(eq, (c_v + inc - eqi) < rem)
            cand_v[pl.ds(i * _L, _L)] = jnp.where(
                jnp.logical_or(lt, take), jnp.int32(1), jnp.int32(0))
            return c_v + plsc.all_reduce_population_count(eq)

        lax.fori_loop(0, _NV, f_body, jnp.zeros((_L,), jnp.int32))

    return pltpu.async_copy(cand_v, out_hbm.at[row], sem_out)


_mesh = plsc.VectorSubcoreMesh(core_axis_name="c", subcore_axis_name="s")


@functools.partial(
    pl.kernel,
    out_type=jax.ShapeDtypeStruct((_B, _N), jnp.int32),
    mesh=_mesh,
    compiler_params=pltpu.CompilerParams(needs_layout_passes=False),
    scratch_types=[
        pltpu.VMEM((_N,), jnp.float32),
        pltpu.VMEM((_N,), jnp.float32),
        pltpu.VMEM((_N,), jnp.int32),
        pltpu.VMEM((_NBINS,), jnp.int32),
        pltpu.VMEM((_B,), jnp.int32),
        pltpu.SemaphoreType.DMA,
        pltpu.SemaphoreType.DMA,
    ],
)
def _topk_mask_kernel(prior_hbm, ks_hbm, out_hbm, data_a, data_b, cand_v,
                      hist_v, ks_v, sem_in, sem_out):
    cid = lax.axis_index("c")
    sid = lax.axis_index("s")
    wid = sid * 2 + cid
    pltpu.sync_copy(ks_hbm, ks_v)
    bufs = (data_a, data_b)
    row0 = wid * _ROWS
    in_cp = pltpu.async_copy(prior_hbm.at[row0], bufs[0], sem_in)
    out_cp = None
    for r in range(_ROWS):
        in_cp.wait()
        if r + 1 < _ROWS:
            in_cp = pltpu.async_copy(
                prior_hbm.at[row0 + r + 1], bufs[(r + 1) % 2], sem_in)
        out_cp = _do_row(out_hbm, row0 + r, bufs[r % 2], cand_v, hist_v,
                         ks_v, sem_out, out_cp)
    out_cp.wait()


def kernel(prior, rates):
    ks = jnp.clip((_N * rates).astype(jnp.int32), 0, _N).reshape(_B)
    out = _topk_mask_kernel(prior, ks)
    return out != 0
